# baseline (speedup 1.0000x reference)
import functools

import jax
import jax.numpy as jnp
from jax import lax
from jax.experimental import pallas as pl
from jax.experimental.pallas import tpu as pltpu
from jax.experimental.pallas import tpu_sc as plsc

V = 8
S = 7
D = 64
B = 16384

NC = 2
NS = 16
L = 16
NW = NC * NS
BPW = B // NW
CHUNK = 128
NCH = BPW // CHUNK
GPC = CHUNK // L


def _body(in_hbm, t1, t2, t3, t4, t5, t6, out_ref,
          in_v, idx_v, gbuf, zbuf, sem, wsem):
    wid = lax.axis_index("s") * NC + lax.axis_index("c")
    base = wid * BPW
    tabs = [t1, t2, t3, t4, t5, t6]

    pltpu.sync_copy(in_hbm.at[pl.ds(0, 6), :, pl.ds(base, BPW)], in_v)

    iota = jax.lax.iota(jnp.int32, L)

    def zero_body(r, _):
        for c in range(D // L):
            zbuf[r, pl.ds(c * L, L)] = jnp.zeros((L,), jnp.float32)
        return _
    lax.fori_loop(0, CHUNK, zero_body, None)

    pending = None
    for j in range(NCH):
        def amax_body(gg, _):
            off = j * CHUNK + gg * L
            e = jnp.zeros((L,), jnp.int32)
            for s in range(S - 1):
                m = in_v[s, 0, pl.ds(off, L)]
                a = jnp.zeros((L,), jnp.int32)
                for v in range(1, V):
                    val = in_v[s, v, pl.ds(off, L)]
                    gt = val > m
                    m = jnp.where(gt, val, m)
                    a = jnp.where(gt, jnp.full((L,), v, jnp.int32), a)
                e = e + a * (V ** s)
                idx_v[s, j, pl.ds(gg * L, L)] = e
            return _
        lax.fori_loop(0, GPC, amax_body, None)

        if pending is not None:
            for w in pending:
                w.wait()

        copies = []
        for d in range(6):
            copies.append(pltpu.make_async_copy(
                tabs[d].at[idx_v.at[d, j]], gbuf.at[d], sem))
        for c in copies:
            c.start()
        for c in copies:
            c.wait()

        rbase = base + j * CHUNK
        writes = [pltpu.make_async_copy(
            zbuf, out_ref.at[pl.ds(rbase, CHUNK), pl.ds(0, D)], wsem)]
        for d in range(6):
            writes.append(pltpu.make_async_copy(
                gbuf.at[d],
                out_ref.at[pl.ds(rbase, CHUNK), pl.ds((d + 1) * D, D)],
                wsem))
        for w in writes:
            w.start()
        pending = writes
    for w in pending:
        w.wait()


_kern = functools.partial(
    pl.kernel,
    out_type=jax.ShapeDtypeStruct((B, S * D), jnp.float32),
    mesh=plsc.VectorSubcoreMesh(core_axis_name="c", subcore_axis_name="s"),
    compiler_params=pltpu.CompilerParams(use_tc_tiling_on_sc=False,
                                         needs_layout_passes=False),
    scratch_types=[
        pltpu.VMEM((6, V, BPW), jnp.float32),
        pltpu.VMEM((6, NCH, CHUNK), jnp.int32),
        pltpu.VMEM((6, CHUNK, D), jnp.float32),
        pltpu.VMEM((CHUNK, D), jnp.float32),
        pltpu.SemaphoreType.DMA,
        pltpu.SemaphoreType.DMA,
    ],
)(_body)


@jax.jit
def _run(inputs2d, t1, t2, t3, t4, t5, t6):
    return _kern(inputs2d, t1, t2, t3, t4, t5, t6).reshape(B, S, D)


def kernel(inputs, table_1, table_2, table_3, table_4, table_5, table_6):
    return _run(inputs.transpose(1, 2, 0),
                table_1, table_2, table_3, table_4, table_5, table_6)


# speedup vs baseline: 1.0007x; 1.0007x over previous
import functools

import jax
import jax.numpy as jnp
from jax import lax
from jax.experimental import pallas as pl
from jax.experimental.pallas import tpu as pltpu
from jax.experimental.pallas import tpu_sc as plsc

V = 8
S = 7
D = 64
B = 16384

NC = 2
NS = 16
L = 16
NW = NC * NS
BPW = B // NW
CHUNK = 128
NCH = BPW // CHUNK
GPC = CHUNK // L


def _body(in_hbm, t1, t2, t3, t4, t5, t6, out_ref,
          in_v, idx_v, gbuf, zbuf, sem, wsem):
    wid = lax.axis_index("s") * NC + lax.axis_index("c")
    base = wid * BPW
    tabs = [t1, t2, t3, t4, t5, t6]

    pltpu.sync_copy(in_hbm.at[pl.ds(0, 6), :, pl.ds(base, BPW)], in_v)

    iota = jax.lax.iota(jnp.int32, L)

    def zero_body(r, _):
        for c in range(D // L):
            zbuf[r, pl.ds(c * L, L)] = jnp.zeros((L,), jnp.float32)
        return _
    lax.fori_loop(0, CHUNK, zero_body, None)

    for j in range(NCH):
        def amax_body(gg, _):
            off = j * CHUNK + gg * L
            e = jnp.zeros((L,), jnp.int32)
            for s in range(S - 1):
                m = in_v[s, 0, pl.ds(off, L)]
                a = jnp.zeros((L,), jnp.int32)
                for v in range(1, V):
                    val = in_v[s, v, pl.ds(off, L)]
                    gt = val > m
                    m = jnp.where(gt, val, m)
                    a = jnp.where(gt, jnp.full((L,), v, jnp.int32), a)
                e = e + a * (V ** s)
                idx_v[s, j, pl.ds(gg * L, L)] = e
            return _
        lax.fori_loop(0, GPC, amax_body, None)

        copies = []
        for d in range(6):
            copies.append(pltpu.make_async_copy(
                tabs[d].at[idx_v.at[d, j]], gbuf.at[d], sem))
        for c in copies:
            c.start()
        for c in copies:
            c.wait()

        rbase = base + j * CHUNK
        writes = [pltpu.make_async_copy(
            zbuf, out_ref.at[pl.ds(rbase, CHUNK), pl.ds(0, D)], wsem)]
        for d in range(6):
            writes.append(pltpu.make_async_copy(
                gbuf.at[d],
                out_ref.at[pl.ds(rbase, CHUNK), pl.ds((d + 1) * D, D)],
                wsem))
        for w in writes:
            w.start()
        for w in writes:
            w.wait()


_kern = functools.partial(
    pl.kernel,
    out_type=jax.ShapeDtypeStruct((B, S * D), jnp.float32),
    mesh=plsc.VectorSubcoreMesh(core_axis_name="c", subcore_axis_name="s"),
    compiler_params=pltpu.CompilerParams(use_tc_tiling_on_sc=False,
                                         needs_layout_passes=False),
    scratch_types=[
        pltpu.VMEM((6, V, BPW), jnp.float32),
        pltpu.VMEM((6, NCH, CHUNK), jnp.int32),
        pltpu.VMEM((6, CHUNK, D), jnp.float32),
        pltpu.VMEM((CHUNK, D), jnp.float32),
        pltpu.SemaphoreType.DMA,
        pltpu.SemaphoreType.DMA,
    ],
)(_body)


@jax.jit
def _run(inputs2d, t1, t2, t3, t4, t5, t6):
    return _kern(inputs2d, t1, t2, t3, t4, t5, t6).reshape(B, S, D)


def kernel(inputs, table_1, table_2, table_3, table_4, table_5, table_6):
    return _run(inputs.transpose(1, 2, 0),
                table_1, table_2, table_3, table_4, table_5, table_6)
